# baseline (speedup 1.0000x reference)
import functools

import jax
import jax.numpy as jnp
from jax import lax
from jax.experimental import pallas as pl
from jax.experimental.pallas import tpu as pltpu
from jax.experimental.pallas import tpu_sc as plsc

_K = 4
_SC_BINS = 1024
_N_SC = 25600
_RB = 80


def _counts_body(x_hbm, out_ref, bufs, sems):
    s = pl.program_id(0)
    nb = pl.num_programs(0)
    K, R, C = bufs.shape
    slot = lax.rem(s, K)

    @pl.when(s == 0)
    def _prologue():
        for k in range(K):
            pltpu.make_async_copy(
                x_hbm.at[pl.ds(k * R, R), :], bufs.at[k], sems.at[k]
            ).start()

    pltpu.make_async_copy(
        x_hbm.at[pl.ds(s * R, R), :], bufs.at[slot], sems.at[slot]
    ).wait()
    xb = bufs[slot]
    m = jnp.max(xb, axis=1, keepdims=True)
    iota = lax.broadcasted_iota(jnp.int32, (R, C), 1).astype(jnp.float32)
    cand = jnp.where(xb == m, iota, jnp.float32(C))
    vote = jnp.min(cand, axis=1, keepdims=True)
    fo = (iota == vote).astype(jnp.bfloat16)
    ones = jnp.ones((1, R), jnp.bfloat16)
    cnt = lax.dot_general(
        ones, fo, (((1,), (0,)), ((), ())),
        preferred_element_type=jnp.float32,
    )

    @pl.when(s == 0)
    def _init():
        out_ref[...] = cnt

    @pl.when(s > 0)
    def _acc():
        out_ref[...] += cnt

    nxt = s + K

    @pl.when(nxt < nb)
    def _issue_next():
        pltpu.make_async_copy(
            x_hbm.at[pl.ds(nxt * R, R), :], bufs.at[slot], sems.at[slot]
        ).start()


def _counts_auto_body(x_ref, out_ref):
    s = pl.program_id(0)
    xb = x_ref[...]
    R, C = xb.shape
    m = jnp.max(xb, axis=1, keepdims=True)
    iota = lax.broadcasted_iota(jnp.int32, (R, C), 1).astype(jnp.float32)
    cand = jnp.where(xb == m, iota, jnp.float32(C))
    vote = jnp.min(cand, axis=1, keepdims=True)
    fo = (iota == vote).astype(jnp.bfloat16)
    ones = jnp.ones((1, R), jnp.bfloat16)
    cnt = lax.dot_general(
        ones, fo, (((1,), (0,)), ((), ())),
        preferred_element_type=jnp.float32,
    )

    @pl.when(s == 0)
    def _init():
        out_ref[...] = cnt

    @pl.when(s > 0)
    def _acc():
        out_ref[...] += cnt


def _tc_counts(x, n_head):
    N, C = x.shape
    R = 1000 if n_head % 1000 == 0 else (800 if n_head % 800 == 0 else n_head)
    grid = n_head // R
    return pl.pallas_call(
        _counts_auto_body,
        grid=(grid,),
        in_specs=[pl.BlockSpec((R, C), lambda i: (i, 0))],
        out_specs=pl.BlockSpec((1, C), lambda i: (0, 0)),
        out_shape=jax.ShapeDtypeStruct((1, C), jnp.float32),
    )(x)


def _sc_body(n_tail, x_hbm, out_hbm, xbuf, votes_v, ones_v, hist_v, hist_sh):
    C = x_hbm.shape[1]
    c = lax.axis_index("c")
    s = lax.axis_index("s")
    wid = s * 2 + c
    W = votes_v.shape[0]
    RB = xbuf.shape[0]
    row0 = x_hbm.shape[0] - n_tail + wid * W
    nblocks = W // RB
    nfull = C // 16
    lane_i = lax.iota(jnp.int32, 16)
    lane = lane_i.astype(jnp.float32)
    neg = jnp.full((16,), -3.4e38, jnp.float32)
    rem = C - nfull * 16

    def _row_argmax(r):
        half = nfull // 2
        bestA = xbuf[r, pl.ds(0, 16)]
        idxA = lane
        bestB = xbuf[r, pl.ds(half * 16, 16)]
        idxB = lane + jnp.float32(half * 16)
        for cc in range(1, half):
            v = xbuf[r, pl.ds(cc * 16, 16)]
            iv = lane + jnp.float32(cc * 16)
            upd = v > bestA
            bestA = jnp.maximum(bestA, v)
            idxA = jnp.where(upd, iv, idxA)
        for cc in range(half + 1, nfull):
            v = xbuf[r, pl.ds(cc * 16, 16)]
            iv = lane + jnp.float32(cc * 16)
            upd = v > bestB
            bestB = jnp.maximum(bestB, v)
            idxB = jnp.where(upd, iv, idxB)
        if rem:
            v = xbuf[r, pl.ds(C - 16, 16)]
            iv = lane + jnp.float32(C - 16)
            upd = v > bestB
            bestB = jnp.maximum(bestB, v)
            idxB = jnp.where(upd, iv, idxB)
        upd = bestB > bestA
        best = jnp.maximum(bestA, bestB)
        bidx = jnp.where(upd, idxB, idxA)
        m = jnp.max(best)
        cands = jnp.where(best == m, bidx, jnp.float32(C))
        return jnp.min(cands)

    def _block(b, carry):
        acc = carry
        pltpu.sync_copy(x_hbm.at[pl.ds(row0 + b * RB, RB), :], xbuf)

        def _row(r, acc):
            vote = _row_argmax(r)
            j = lax.rem(r, 16)
            mask = lax.iota(jnp.int32, 16) == j
            acc = jnp.where(mask, jnp.full((16,), 1, jnp.float32) * vote, acc)

            @pl.when(j == 15)
            def _store():
                g = lax.div(r, 16)
                votes_v[pl.ds(b * RB + g * 16, 16)] = acc.astype(jnp.int32)

            return acc

        return lax.fori_loop(0, RB, _row, acc)

    lax.fori_loop(0, nblocks, _block, neg)

    def _fill_ones(i, carry):
        ones_v[pl.ds(i * 16, 16)] = jnp.full((16,), 1, jnp.int32)
        return carry

    lax.fori_loop(0, W // 16, _fill_ones, 0)

    def _zero_hist(i, carry):
        hist_v[pl.ds(i * 16, 16)] = jnp.zeros((16,), jnp.int32)
        return carry

    lax.fori_loop(0, _SC_BINS // 16, _zero_hist, 0)

    @pl.when(s == 0)
    def _zero_shared():
        pltpu.sync_copy(hist_v, hist_sh)

    plsc.subcore_barrier()
    pltpu.sync_copy(ones_v, hist_sh.at[votes_v], add=True)
    plsc.subcore_barrier()

    @pl.when(s == 0)
    def _writeback():
        pltpu.sync_copy(hist_sh, hist_v)
        pltpu.sync_copy(hist_v, out_hbm.at[c])


def _sc_counts(x, n_tail):
    N, C = x.shape
    W = n_tail // 32
    mesh = plsc.VectorSubcoreMesh(core_axis_name="c", subcore_axis_name="s")
    k = functools.partial(
        pl.kernel,
        out_type=jax.ShapeDtypeStruct((2, _SC_BINS), jnp.int32),
        mesh=mesh,
        scratch_types=[
            pltpu.VMEM((_RB, C), jnp.float32),
            pltpu.VMEM((W,), jnp.int32),
            pltpu.VMEM((W,), jnp.int32),
            pltpu.VMEM((_SC_BINS,), jnp.int32),
            pltpu.VMEM_SHARED((_SC_BINS,), jnp.int32),
        ],
        compiler_params=pltpu.CompilerParams(
            use_tc_tiling_on_sc=True, needs_layout_passes=False),
        cost_estimate=pl.CostEstimate(
            flops=2 * n_tail * C,
            transcendentals=0,
            bytes_accessed=4 * n_tail * C,
        ),
    )(functools.partial(_sc_body, n_tail))
    return k(x)


def _merge_body(tc_ref, part_ref, out_ref):
    C = out_ref.shape[1]
    sc_counts = (part_ref[0, 0:C] + part_ref[1, 0:C]).astype(jnp.float32)
    counts = tc_ref[0, :] + sc_counts
    cm = jnp.max(counts)
    iota = lax.iota(jnp.int32, C).astype(jnp.float32)
    cand = jnp.where(counts == cm, iota, jnp.float32(C))
    w = jnp.min(cand)
    out_ref[0, :] = (iota == w).astype(jnp.int32)


def _tc_merge(tc_counts, parts, C):
    out = pl.pallas_call(
        _merge_body,
        out_shape=jax.ShapeDtypeStruct((1, C), jnp.int32),
    )(tc_counts, parts)
    return out[0]


def kernel(x):
    N, C = x.shape
    n_tail = _N_SC if (N == 100000 and C == 1000) else 0
    if n_tail:
        parts = _sc_counts(x, n_tail)
    else:
        parts = jnp.zeros((2, _SC_BINS), jnp.int32)
    tc_counts = _tc_counts(x, N - n_tail)
    return _tc_merge(tc_counts, parts, C)


# speedup vs baseline: 1.0588x; 1.0588x over previous
import functools

import jax
import jax.numpy as jnp
from jax import lax
from jax.experimental import pallas as pl
from jax.experimental.pallas import tpu as pltpu
from jax.experimental.pallas import tpu_sc as plsc

_K = 4
_SC_BINS = 1024
_N_SC = 12800
_RB = 80


def _counts_body(x_hbm, out_ref, bufs, sems):
    s = pl.program_id(0)
    nb = pl.num_programs(0)
    K, R, C = bufs.shape
    slot = lax.rem(s, K)

    @pl.when(s == 0)
    def _prologue():
        for k in range(K):
            pltpu.make_async_copy(
                x_hbm.at[pl.ds(k * R, R), :], bufs.at[k], sems.at[k]
            ).start()

    pltpu.make_async_copy(
        x_hbm.at[pl.ds(s * R, R), :], bufs.at[slot], sems.at[slot]
    ).wait()
    xb = bufs[slot]
    m = jnp.max(xb, axis=1, keepdims=True)
    iota = lax.broadcasted_iota(jnp.int32, (R, C), 1).astype(jnp.float32)
    cand = jnp.where(xb == m, iota, jnp.float32(C))
    vote = jnp.min(cand, axis=1, keepdims=True)
    fo = (iota == vote).astype(jnp.bfloat16)
    ones = jnp.ones((1, R), jnp.bfloat16)
    cnt = lax.dot_general(
        ones, fo, (((1,), (0,)), ((), ())),
        preferred_element_type=jnp.float32,
    )

    @pl.when(s == 0)
    def _init():
        out_ref[...] = cnt

    @pl.when(s > 0)
    def _acc():
        out_ref[...] += cnt

    nxt = s + K

    @pl.when(nxt < nb)
    def _issue_next():
        pltpu.make_async_copy(
            x_hbm.at[pl.ds(nxt * R, R), :], bufs.at[slot], sems.at[slot]
        ).start()


def _tc_counts(x, n_head):
    N, C = x.shape
    R = 1000 if n_head % 1000 == 0 else (800 if n_head % 800 == 0 else n_head)
    grid = n_head // R
    ring = min(_K, grid)
    return pl.pallas_call(
        _counts_body,
        grid=(grid,),
        in_specs=[pl.BlockSpec(memory_space=pltpu.HBM)],
        out_specs=pl.BlockSpec((1, C), lambda i: (0, 0)),
        out_shape=jax.ShapeDtypeStruct((1, C), jnp.float32),
        scratch_shapes=[
            pltpu.VMEM((ring, R, C), jnp.float32),
            pltpu.SemaphoreType.DMA((ring,)),
        ],
    )(x)


def _sc_body(n_tail, x_hbm, out_hbm, xbuf, votes_v, ones_v, hist_v, hist_sh):
    C = x_hbm.shape[1]
    c = lax.axis_index("c")
    s = lax.axis_index("s")
    wid = s * 2 + c
    W = votes_v.shape[0]
    RB = xbuf.shape[0]
    row0 = x_hbm.shape[0] - n_tail + wid * W
    nblocks = W // RB
    nfull = C // 16
    lane_i = lax.iota(jnp.int32, 16)
    lane = lane_i.astype(jnp.float32)
    neg = jnp.full((16,), -3.4e38, jnp.float32)
    rem = C - nfull * 16

    def _row_argmax(r):
        half = nfull // 2
        bestA = xbuf[r, pl.ds(0, 16)]
        idxA = lane
        bestB = xbuf[r, pl.ds(half * 16, 16)]
        idxB = lane + jnp.float32(half * 16)
        for cc in range(1, half):
            v = xbuf[r, pl.ds(cc * 16, 16)]
            iv = lane + jnp.float32(cc * 16)
            upd = v > bestA
            bestA = jnp.maximum(bestA, v)
            idxA = jnp.where(upd, iv, idxA)
        for cc in range(half + 1, nfull):
            v = xbuf[r, pl.ds(cc * 16, 16)]
            iv = lane + jnp.float32(cc * 16)
            upd = v > bestB
            bestB = jnp.maximum(bestB, v)
            idxB = jnp.where(upd, iv, idxB)
        if rem:
            v = xbuf[r, pl.ds(C - 16, 16)]
            iv = lane + jnp.float32(C - 16)
            upd = v > bestB
            bestB = jnp.maximum(bestB, v)
            idxB = jnp.where(upd, iv, idxB)
        upd = bestB > bestA
        best = jnp.maximum(bestA, bestB)
        bidx = jnp.where(upd, idxB, idxA)
        m = jnp.max(best)
        cands = jnp.where(best == m, bidx, jnp.float32(C))
        return jnp.min(cands)

    def _block(b, carry):
        acc = carry
        pltpu.sync_copy(x_hbm.at[pl.ds(row0 + b * RB, RB), :], xbuf)

        def _row(r, acc):
            vote = _row_argmax(r)
            j = lax.rem(r, 16)
            mask = lax.iota(jnp.int32, 16) == j
            acc = jnp.where(mask, jnp.full((16,), 1, jnp.float32) * vote, acc)

            @pl.when(j == 15)
            def _store():
                g = lax.div(r, 16)
                votes_v[pl.ds(b * RB + g * 16, 16)] = acc.astype(jnp.int32)

            return acc

        return lax.fori_loop(0, RB, _row, acc)

    lax.fori_loop(0, nblocks, _block, neg)

    def _fill_ones(i, carry):
        ones_v[pl.ds(i * 16, 16)] = jnp.full((16,), 1, jnp.int32)
        return carry

    lax.fori_loop(0, W // 16, _fill_ones, 0)

    def _zero_hist(i, carry):
        hist_v[pl.ds(i * 16, 16)] = jnp.zeros((16,), jnp.int32)
        return carry

    lax.fori_loop(0, _SC_BINS // 16, _zero_hist, 0)

    @pl.when(s == 0)
    def _zero_shared():
        pltpu.sync_copy(hist_v, hist_sh)

    plsc.subcore_barrier()
    pltpu.sync_copy(ones_v, hist_sh.at[votes_v], add=True)
    plsc.subcore_barrier()

    @pl.when(s == 0)
    def _writeback():
        pltpu.sync_copy(hist_sh, hist_v)
        pltpu.sync_copy(hist_v, out_hbm.at[c])


def _sc_counts(x, n_tail):
    N, C = x.shape
    W = n_tail // 32
    mesh = plsc.VectorSubcoreMesh(core_axis_name="c", subcore_axis_name="s")
    k = functools.partial(
        pl.kernel,
        out_type=jax.ShapeDtypeStruct((2, _SC_BINS), jnp.int32),
        mesh=mesh,
        scratch_types=[
            pltpu.VMEM((_RB, C), jnp.float32),
            pltpu.VMEM((W,), jnp.int32),
            pltpu.VMEM((W,), jnp.int32),
            pltpu.VMEM((_SC_BINS,), jnp.int32),
            pltpu.VMEM_SHARED((_SC_BINS,), jnp.int32),
        ],
        compiler_params=pltpu.CompilerParams(
            use_tc_tiling_on_sc=True, needs_layout_passes=False),
        cost_estimate=pl.CostEstimate(
            flops=2 * n_tail * C,
            transcendentals=0,
            bytes_accessed=4 * n_tail * C,
        ),
    )(functools.partial(_sc_body, n_tail))
    return k(x)


def _merge_body(tc_ref, part_ref, out_ref):
    C = out_ref.shape[1]
    sc_counts = (part_ref[0, 0:C] + part_ref[1, 0:C]).astype(jnp.float32)
    counts = tc_ref[0, :] + sc_counts
    cm = jnp.max(counts)
    iota = lax.iota(jnp.int32, C).astype(jnp.float32)
    cand = jnp.where(counts == cm, iota, jnp.float32(C))
    w = jnp.min(cand)
    out_ref[0, :] = (iota == w).astype(jnp.int32)


def _tc_merge(tc_counts, parts, C):
    out = pl.pallas_call(
        _merge_body,
        out_shape=jax.ShapeDtypeStruct((1, C), jnp.int32),
    )(tc_counts, parts)
    return out[0]


def kernel(x):
    N, C = x.shape
    n_tail = _N_SC if (N == 100000 and C == 1000) else 0
    if n_tail:
        parts = _sc_counts(x, n_tail)
    else:
        parts = jnp.zeros((2, _SC_BINS), jnp.int32)
    tc_counts = _tc_counts(x, N - n_tail)
    return _tc_merge(tc_counts, parts, C)
